# baseline (speedup 1.0000x reference)
import jax
import jax.numpy as jnp
from jax import lax
from jax.experimental import pallas as pl
from jax.experimental.pallas import tpu as pltpu
from jax.experimental.pallas import tpu_sc as plsc

N = 10000
E = 160000
H = 256
NP = 10240
EP = 163840
HALF = NP // 2
NDUMP = 256
F = 512
HC = 128
NCH = F // HC

_mesh = plsc.VectorSubcoreMesh(
    core_axis_name="c", subcore_axis_name="s", num_cores=2, num_subcores=16
)


def _deg_body(ed_hbm, ones_hbm, out_hbm, src_v, dst_v, ones_v, zero_v,
              dego_sm, degi_sm):
    cid = lax.axis_index("c")
    sid = lax.axis_index("s")
    w = cid * 16 + sid

    for k in range(40):
        zero_v[pl.ds(k * 16, 16)] = jnp.zeros((16,), jnp.float32)

    pltpu.sync_copy(zero_v, dego_sm.at[pl.ds(sid * 640, 640)])
    pltpu.sync_copy(zero_v, degi_sm.at[pl.ds(sid * 640, 640)])

    pltpu.sync_copy(ed_hbm.at[0, w], src_v)
    pltpu.sync_copy(ed_hbm.at[1, w], dst_v)
    pltpu.sync_copy(ones_hbm, ones_v)

    plsc.subcore_barrier()

    pltpu.sync_copy(ones_v, dego_sm.at[src_v], add=True)
    pltpu.sync_copy(ones_v, degi_sm.at[dst_v], add=True)

    plsc.subcore_barrier()

    pltpu.sync_copy(dego_sm.at[pl.ds(sid * 640, 640)],
                    out_hbm.at[cid, 0, pl.ds(sid * 640, 640)])
    pltpu.sync_copy(degi_sm.at[pl.ds(sid * 640, 640)],
                    out_hbm.at[cid, 1, pl.ds(sid * 640, 640)])


_deg_kernel = pl.kernel(
    _deg_body,
    out_type=jax.ShapeDtypeStruct((2, 2, NP), jnp.float32),
    mesh=_mesh,
    scratch_types=[
        pltpu.VMEM((5120,), jnp.int32),
        pltpu.VMEM((5120,), jnp.int32),
        pltpu.VMEM((5120,), jnp.float32),
        pltpu.VMEM((640,), jnp.float32),
        pltpu.VMEM_SHARED((NP,), jnp.float32),
        pltpu.VMEM_SHARED((NP,), jnp.float32),
    ],
)


AR = 10112
SR = AR // 16


def _edge_body(x0_hbm, x1_hbm, x2_hbm, x3_hbm, ed_hbm, z_hbm,
               o0_hbm, o1_hbm, o2_hbm, o3_hbm,
               idx_v, rows_a, rows_b, acc_sm, gsa, gsb):
    cid = lax.axis_index("c")
    sid = lax.axis_index("s")

    def _run_chunk(x_hbm, o_hbm):
        pltpu.sync_copy(z_hbm, acc_sm.at[pl.ds(sid * SR, SR)])
        plsc.subcore_barrier()

        for h in range(2):
            pltpu.sync_copy(ed_hbm.at[0, sid, h], idx_v.at[pl.ds(0, 40)])
            pltpu.sync_copy(ed_hbm.at[1, sid, h], idx_v.at[pl.ds(40, 40)])

            pltpu.async_copy(x_hbm.at[idx_v.at[0]], rows_a, gsa)

            def _edge(t, _):
                j = 2 * t
                pltpu.make_async_copy(x_hbm.at[idx_v.at[j]], rows_a,
                                      gsa).wait()
                pltpu.async_copy(x_hbm.at[idx_v.at[j + 1]], rows_b, gsb)
                pltpu.sync_copy(rows_a, acc_sm.at[idx_v.at[40 + j]], add=True)
                pltpu.make_async_copy(x_hbm.at[idx_v.at[j + 1]], rows_b,
                                      gsb).wait()

                @pl.when(t < 19)
                def _():
                    pltpu.async_copy(x_hbm.at[idx_v.at[j + 2]], rows_a, gsa)

                pltpu.sync_copy(rows_b, acc_sm.at[idx_v.at[41 + j]], add=True)
                return 0

            lax.fori_loop(0, 20, _edge, 0)

        plsc.subcore_barrier()

        pltpu.sync_copy(acc_sm.at[pl.ds(sid * SR, SR)],
                        o_hbm.at[pl.ds(sid * SR, SR)])

    for k in range(2):
        @pl.when(cid == 0)
        def _():
            _run_chunk((x0_hbm, x1_hbm)[k], (o0_hbm, o1_hbm)[k])

        @pl.when(cid == 1)
        def _():
            _run_chunk((x2_hbm, x3_hbm)[k], (o2_hbm, o3_hbm)[k])


_edge_kernel = pl.kernel(
    _edge_body,
    out_type=[jax.ShapeDtypeStruct((NP, HC), jnp.float32)
              for _ in range(NCH)],
    mesh=_mesh,
    scratch_types=[
        pltpu.VMEM((80, 128), jnp.int32),
        pltpu.VMEM((128, HC), jnp.float32),
        pltpu.VMEM((128, HC), jnp.float32),
        pltpu.VMEM_SHARED((AR, HC), jnp.float32),
        pltpu.SemaphoreType.DMA,
        pltpu.SemaphoreType.DMA,
    ],
)


NB = 1024
NBLK = NP // NB


def _tc1_body(q_ref, p_ref, wq_ref, wp_ref, deg_ref, x0_ref, x1_ref, x2_ref,
              x3_ref):
    i = pl.program_id(0)
    d = deg_ref[0, 0, pl.ds(i * NB, NB)] + deg_ref[1, 0, pl.ds(i * NB, NB)]
    no = jnp.where(d > 0, lax.rsqrt(d), 0.0).reshape(NB, 1)
    xq = jnp.dot(q_ref[...].astype(jnp.bfloat16),
                 wq_ref[...].astype(jnp.bfloat16),
                 preferred_element_type=jnp.float32) * no
    xp = jnp.dot(p_ref[...].astype(jnp.bfloat16),
                 wp_ref[...].astype(jnp.bfloat16),
                 preferred_element_type=jnp.float32) * no
    x0_ref[...] = xq[:, 0:HC]
    x1_ref[...] = xq[:, HC:H]
    x2_ref[...] = xp[:, 0:HC]
    x3_ref[...] = xp[:, HC:H]


def _tc2_body(a0_ref, a1_ref, a2_ref, a3_ref, deg_ref, bqp_ref, w1_ref,
              b1_ref, w2_ref, b2_ref, out_ref):
    i = pl.program_id(0)
    d = deg_ref[0, 1, pl.ds(i * NB, NB)] + deg_ref[1, 1, pl.ds(i * NB, NB)]
    ni = jnp.where(d > 0, lax.rsqrt(d), 0.0).reshape(NB, 1)
    pre = b1_ref[...]
    for k, a_ref in enumerate((a0_ref, a1_ref, a2_ref, a3_ref)):
        zk = a_ref[...] * ni + bqp_ref[:, k * HC:(k + 1) * HC]
        pre = pre + jnp.dot(zk.astype(jnp.bfloat16),
                            w1_ref[k * HC:(k + 1) * HC].astype(jnp.bfloat16),
                            preferred_element_type=jnp.float32)
    h = jnp.tanh(pre)
    y = jnp.dot(h.astype(jnp.bfloat16), w2_ref[...].astype(jnp.bfloat16),
                preferred_element_type=jnp.float32) + b2_ref[...]
    row = i * NB + lax.broadcasted_iota(jnp.int32, (NB, 1), 0)
    s = jnp.sum(jnp.where(row < N, y * y, 0.0))

    @pl.when(i == 0)
    def _():
        out_ref[0, 0] = 0.0

    out_ref[0, 0] += 0.5 * s


def kernel(q, p, edge_index, Wq, bq, Wp, bp, W1, b1, W2, b2):
    qpad = jnp.pad(q, ((0, NP - N), (0, 0)))
    ppad = jnp.pad(p, ((0, NP - N), (0, 0)))
    ar = jnp.arange(EP - E, dtype=jnp.int32)
    pad_hi = N + ar % (NP - N)
    edd = jnp.concatenate([edge_index, jnp.stack([pad_hi, pad_hi])], axis=1)
    ede = jnp.concatenate([edge_index, jnp.stack([pad_hi, ar % N])], axis=1)
    ed32 = edd.reshape(2, 32, 5120)
    ed16 = ede.reshape(2, 16, 2, 40, 128)
    zrows = jnp.zeros((SR, HC), jnp.float32)
    ones5k = jnp.ones((5120,), jnp.float32)
    bqp = jnp.concatenate([bq, bp]).reshape(1, F)

    degp = _deg_kernel(ed32, ones5k)

    x4 = pl.pallas_call(
        _tc1_body,
        grid=(NBLK,),
        in_specs=[
            pl.BlockSpec((NB, H), lambda i: (i, 0)),
            pl.BlockSpec((NB, H), lambda i: (i, 0)),
            pl.BlockSpec((H, H), lambda i: (0, 0)),
            pl.BlockSpec((H, H), lambda i: (0, 0)),
            pl.BlockSpec((2, 2, NP), lambda i: (0, 0, 0)),
        ],
        out_specs=[pl.BlockSpec((NB, HC), lambda i: (i, 0))
                   for _ in range(NCH)],
        out_shape=[jax.ShapeDtypeStruct((NP, HC), jnp.float32)
                   for _ in range(NCH)],
    )(qpad, ppad, Wq, Wp, degp)

    agg0, agg1, agg2, agg3 = _edge_kernel(x4[0], x4[1], x4[2], x4[3],
                                          ed16, zrows)

    hs = pl.pallas_call(
        _tc2_body,
        grid=(NBLK,),
        in_specs=[pl.BlockSpec((NB, HC), lambda i: (i, 0))
                  for _ in range(NCH)] + [
            pl.BlockSpec((2, 2, NP), lambda i: (0, 0, 0)),
            pl.BlockSpec((1, F), lambda i: (0, 0)),
            pl.BlockSpec((F, H), lambda i: (0, 0)),
            pl.BlockSpec((H,), lambda i: (0,)),
            pl.BlockSpec((H, H), lambda i: (0, 0)),
            pl.BlockSpec((H,), lambda i: (0,)),
        ],
        out_specs=pl.BlockSpec((1, 1), lambda i: (0, 0),
                               memory_space=pltpu.SMEM),
        out_shape=jax.ShapeDtypeStruct((1, 1), jnp.float32),
    )(agg0, agg1, agg2, agg3, degp, bqp, W1, b1, W2, b2)

    return hs[0, 0]


# speedup vs baseline: 1.0104x; 1.0104x over previous
import jax
import jax.numpy as jnp
from jax import lax
from jax.experimental import pallas as pl
from jax.experimental.pallas import tpu as pltpu
from jax.experimental.pallas import tpu_sc as plsc

N = 10000
E = 160000
H = 256
NP = 10240
EP = 163840
HALF = NP // 2
NDUMP = 256
F = 512
HC = 128
NCH = F // HC

_mesh = plsc.VectorSubcoreMesh(
    core_axis_name="c", subcore_axis_name="s", num_cores=2, num_subcores=16
)


def _deg_body(ed_hbm, out_hbm, src_v, dst_v, ones_v, zero_v,
              dego_sm, degi_sm):
    cid = lax.axis_index("c")
    sid = lax.axis_index("s")
    w = cid * 16 + sid

    for k in range(40):
        zero_v[pl.ds(k * 16, 16)] = jnp.zeros((16,), jnp.float32)

    pltpu.sync_copy(zero_v, dego_sm.at[pl.ds(sid * 640, 640)])
    pltpu.sync_copy(zero_v, degi_sm.at[pl.ds(sid * 640, 640)])

    pltpu.sync_copy(ed_hbm.at[0, w], src_v)
    pltpu.sync_copy(ed_hbm.at[1, w], dst_v)

    def _ones(j, _):
        ones_v[pl.ds(j * 16, 16)] = jnp.full((16,), 1.0, jnp.float32)
        return 0

    lax.fori_loop(0, 320, _ones, 0)

    plsc.subcore_barrier()

    pltpu.sync_copy(ones_v, dego_sm.at[src_v], add=True)
    pltpu.sync_copy(ones_v, degi_sm.at[dst_v], add=True)

    plsc.subcore_barrier()

    pltpu.sync_copy(dego_sm.at[pl.ds(sid * 640, 640)],
                    out_hbm.at[cid, 0, pl.ds(sid * 640, 640)])
    pltpu.sync_copy(degi_sm.at[pl.ds(sid * 640, 640)],
                    out_hbm.at[cid, 1, pl.ds(sid * 640, 640)])


_deg_kernel = pl.kernel(
    _deg_body,
    out_type=jax.ShapeDtypeStruct((2, 2, NP), jnp.float32),
    mesh=_mesh,
    scratch_types=[
        pltpu.VMEM((5120,), jnp.int32),
        pltpu.VMEM((5120,), jnp.int32),
        pltpu.VMEM((5120,), jnp.float32),
        pltpu.VMEM((640,), jnp.float32),
        pltpu.VMEM_SHARED((NP,), jnp.float32),
        pltpu.VMEM_SHARED((NP,), jnp.float32),
    ],
)


AR = 10112
SR = AR // 16


def _edge_body(x0_hbm, x1_hbm, x2_hbm, x3_hbm, ed_hbm, z_hbm,
               o0_hbm, o1_hbm, o2_hbm, o3_hbm,
               idx_v, rows_a, rows_b, acc_sm, gsa, gsb):
    cid = lax.axis_index("c")
    sid = lax.axis_index("s")

    def _run_chunk(x_hbm, o_hbm):
        pltpu.sync_copy(z_hbm, acc_sm.at[pl.ds(sid * SR, SR)])
        plsc.subcore_barrier()

        for h in range(2):
            pltpu.sync_copy(ed_hbm.at[0, sid, h], idx_v.at[pl.ds(0, 40)])
            pltpu.sync_copy(ed_hbm.at[1, sid, h], idx_v.at[pl.ds(40, 40)])

            pltpu.async_copy(x_hbm.at[idx_v.at[0]], rows_a, gsa)

            def _edge(t, _):
                j = 2 * t
                pltpu.make_async_copy(x_hbm.at[idx_v.at[j]], rows_a,
                                      gsa).wait()
                pltpu.async_copy(x_hbm.at[idx_v.at[j + 1]], rows_b, gsb)
                pltpu.sync_copy(rows_a, acc_sm.at[idx_v.at[40 + j]], add=True)
                pltpu.make_async_copy(x_hbm.at[idx_v.at[j + 1]], rows_b,
                                      gsb).wait()

                @pl.when(t < 19)
                def _():
                    pltpu.async_copy(x_hbm.at[idx_v.at[j + 2]], rows_a, gsa)

                pltpu.sync_copy(rows_b, acc_sm.at[idx_v.at[41 + j]], add=True)
                return 0

            lax.fori_loop(0, 20, _edge, 0)

        plsc.subcore_barrier()

        pltpu.sync_copy(acc_sm.at[pl.ds(sid * SR, SR)],
                        o_hbm.at[pl.ds(sid * SR, SR)])

    for k in range(2):
        @pl.when(cid == 0)
        def _():
            _run_chunk((x0_hbm, x1_hbm)[k], (o0_hbm, o1_hbm)[k])

        @pl.when(cid == 1)
        def _():
            _run_chunk((x2_hbm, x3_hbm)[k], (o2_hbm, o3_hbm)[k])


_edge_kernel = pl.kernel(
    _edge_body,
    out_type=[jax.ShapeDtypeStruct((NP, HC), jnp.float32)
              for _ in range(NCH)],
    mesh=_mesh,
    scratch_types=[
        pltpu.VMEM((80, 128), jnp.int32),
        pltpu.VMEM((128, HC), jnp.float32),
        pltpu.VMEM((128, HC), jnp.float32),
        pltpu.VMEM_SHARED((AR, HC), jnp.float32),
        pltpu.SemaphoreType.DMA,
        pltpu.SemaphoreType.DMA,
    ],
)


NB = 1024
NBLK = NP // NB


def _tc1_body(q_ref, p_ref, wq_ref, wp_ref, deg_ref, x0_ref, x1_ref, x2_ref,
              x3_ref):
    i = pl.program_id(0)
    d = deg_ref[0, 0, pl.ds(i * NB, NB)] + deg_ref[1, 0, pl.ds(i * NB, NB)]
    no = jnp.where(d > 0, lax.rsqrt(d), 0.0).reshape(NB, 1)
    xq = jnp.dot(q_ref[...].astype(jnp.bfloat16),
                 wq_ref[...].astype(jnp.bfloat16),
                 preferred_element_type=jnp.float32) * no
    xp = jnp.dot(p_ref[...].astype(jnp.bfloat16),
                 wp_ref[...].astype(jnp.bfloat16),
                 preferred_element_type=jnp.float32) * no
    x0_ref[...] = xq[:, 0:HC]
    x1_ref[...] = xq[:, HC:H]
    x2_ref[...] = xp[:, 0:HC]
    x3_ref[...] = xp[:, HC:H]


def _tc2_body(a0_ref, a1_ref, a2_ref, a3_ref, deg_ref, bqp_ref, w1_ref,
              b1_ref, w2_ref, b2_ref, out_ref):
    i = pl.program_id(0)
    d = deg_ref[0, 1, pl.ds(i * NB, NB)] + deg_ref[1, 1, pl.ds(i * NB, NB)]
    ni = jnp.where(d > 0, lax.rsqrt(d), 0.0).reshape(NB, 1)
    pre = b1_ref[...]
    for k, a_ref in enumerate((a0_ref, a1_ref, a2_ref, a3_ref)):
        zk = a_ref[...] * ni + bqp_ref[:, k * HC:(k + 1) * HC]
        pre = pre + jnp.dot(zk.astype(jnp.bfloat16),
                            w1_ref[k * HC:(k + 1) * HC].astype(jnp.bfloat16),
                            preferred_element_type=jnp.float32)
    h = jnp.tanh(pre)
    y = jnp.dot(h.astype(jnp.bfloat16), w2_ref[...].astype(jnp.bfloat16),
                preferred_element_type=jnp.float32) + b2_ref[...]
    row = i * NB + lax.broadcasted_iota(jnp.int32, (NB, 1), 0)
    s = jnp.sum(jnp.where(row < N, y * y, 0.0))

    @pl.when(i == 0)
    def _():
        out_ref[0, 0] = 0.0

    out_ref[0, 0] += 0.5 * s


def kernel(q, p, edge_index, Wq, bq, Wp, bp, W1, b1, W2, b2):
    qpad = jnp.pad(q, ((0, NP - N), (0, 0)))
    ppad = jnp.pad(p, ((0, NP - N), (0, 0)))
    ar = jnp.arange(EP - E, dtype=jnp.int32)
    pad_hi = N + ar % (NP - N)
    edd = jnp.concatenate([edge_index, jnp.stack([pad_hi, pad_hi])], axis=1)
    ede = jnp.concatenate([edge_index, jnp.stack([pad_hi, ar % N])], axis=1)
    ed32 = edd.reshape(2, 32, 5120)
    ed16 = ede.reshape(2, 16, 2, 40, 128)
    zrows = jnp.zeros((SR, HC), jnp.float32)
    bqp = jnp.concatenate([bq, bp]).reshape(1, F)

    degp = _deg_kernel(ed32)

    x4 = pl.pallas_call(
        _tc1_body,
        grid=(NBLK,),
        in_specs=[
            pl.BlockSpec((NB, H), lambda i: (i, 0)),
            pl.BlockSpec((NB, H), lambda i: (i, 0)),
            pl.BlockSpec((H, H), lambda i: (0, 0)),
            pl.BlockSpec((H, H), lambda i: (0, 0)),
            pl.BlockSpec((2, 2, NP), lambda i: (0, 0, 0)),
        ],
        out_specs=[pl.BlockSpec((NB, HC), lambda i: (i, 0))
                   for _ in range(NCH)],
        out_shape=[jax.ShapeDtypeStruct((NP, HC), jnp.float32)
                   for _ in range(NCH)],
    )(qpad, ppad, Wq, Wp, degp)

    agg0, agg1, agg2, agg3 = _edge_kernel(x4[0], x4[1], x4[2], x4[3],
                                          ed16, zrows)

    hs = pl.pallas_call(
        _tc2_body,
        grid=(NBLK,),
        in_specs=[pl.BlockSpec((NB, HC), lambda i: (i, 0))
                  for _ in range(NCH)] + [
            pl.BlockSpec((2, 2, NP), lambda i: (0, 0, 0)),
            pl.BlockSpec((1, F), lambda i: (0, 0)),
            pl.BlockSpec((F, H), lambda i: (0, 0)),
            pl.BlockSpec((H,), lambda i: (0,)),
            pl.BlockSpec((H, H), lambda i: (0, 0)),
            pl.BlockSpec((H,), lambda i: (0,)),
        ],
        out_specs=pl.BlockSpec((1, 1), lambda i: (0, 0),
                               memory_space=pltpu.SMEM),
        out_shape=jax.ShapeDtypeStruct((1, 1), jnp.float32),
    )(agg0, agg1, agg2, agg3, degp, bqp, W1, b1, W2, b2)

    return hs[0, 0]
